# baseline (speedup 1.0000x reference)
import jax
import jax.numpy as jnp
from jax.experimental import pallas as pl
from jax.experimental.pallas import tpu as pltpu

_HEADS = 12
_DH = 64
_EXPERTS = 8
_HIDDEN = 3072
_RB = 256
_LN_EPS = 1e-5
_SCALE = _DH ** -0.5
_NEG = -1e30
_BF = jnp.bfloat16
_F32 = jnp.float32


def _ln(xb, g, b):
    m = jnp.mean(xb, axis=-1, keepdims=True)
    v = jnp.mean((xb - m) ** 2, axis=-1, keepdims=True)
    return (xb - m) * jax.lax.rsqrt(v + _LN_EPS) * g + b


def _ln_qkv_body(x_ref, g_ref, b_ref, w_ref, wb_ref, out_ref):
    h = _ln(x_ref[...], g_ref[...], b_ref[...])
    out_ref[...] = (
        jnp.dot(h.astype(_BF), w_ref[...], preferred_element_type=_F32)
        + wb_ref[...]
    )


def _attn_body(q_ref, k_ref, v_ref, o_ref):
    q = q_ref[...].astype(_BF)
    k = k_ref[...].astype(_BF)
    v = v_ref[...].astype(_BF)
    outs = []
    for i in range(2):
        qp = q[:, i * _DH:(i + 1) * _DH]
        kp = k[:, i * _DH:(i + 1) * _DH]
        vp = v[:, i * _DH:(i + 1) * _DH]
        s = jax.lax.dot_general(
            qp, kp, (((1,), (1,)), ((), ())), preferred_element_type=_F32
        ) * _SCALE
        m = jnp.max(s, axis=-1, keepdims=True)
        p = jnp.exp(s - m)
        p = p / jnp.sum(p, axis=-1, keepdims=True)
        outs.append(jnp.dot(p.astype(_BF), vp, preferred_element_type=_F32))
    o_ref[...] = jnp.concatenate(outs, axis=-1)


def _proj_gate_body(x_ref, o_ref, pw_ref, pb_ref, g2_ref, b2_ref, wg_ref,
                    xr_ref, nx_ref, idx_ref, gate_ref):
    xr = x_ref[...] + jnp.dot(
        o_ref[...].astype(_BF), pw_ref[...], preferred_element_type=_F32
    ) + pb_ref[...]
    xr_ref[...] = xr
    nx = _ln(xr, g2_ref[...], b2_ref[...])
    nxb = nx.astype(_BF)
    nx_ref[...] = nxb
    logits = jnp.dot(nxb, wg_ref[...], preferred_element_type=_F32)
    ii = jax.lax.broadcasted_iota(jnp.int32, logits.shape, 1)
    m1 = jnp.max(logits, axis=-1, keepdims=True)
    i1 = jnp.min(jnp.where(logits == m1, ii, _EXPERTS), axis=-1, keepdims=True)
    l2 = jnp.where(ii == i1, _NEG, logits)
    m2 = jnp.max(l2, axis=-1, keepdims=True)
    i2 = jnp.min(jnp.where(l2 == m2, ii, _EXPERTS), axis=-1, keepdims=True)
    d = jnp.exp(m2 - m1)
    g1 = 1.0 / (1.0 + d)
    g2 = d / (1.0 + d)
    idx_ref[...] = jnp.concatenate([i1, i2], axis=-1)
    gate_ref[...] = jnp.concatenate([g1, g2], axis=-1)


def _fiota(shape, d):
    return jax.lax.broadcasted_iota(jnp.int32, shape, d).astype(_F32)


def _route_a_body(er_ref, ec_ref, pos_ref, be_ref, bv_ref, cnt_ref):
    c = pl.program_id(0)
    nb = be_ref.shape[1]
    e8 = _fiota((1, _EXPERTS), 1)
    onehot_all = (ec_ref[...] == e8).astype(_F32)
    counts = jnp.sum(onehot_all, axis=0, keepdims=True)
    padded = jnp.floor((counts + (_RB - 1)) * (1.0 / _RB)) * _RB
    tril8 = (_fiota((_EXPERTS, _EXPERTS), 0)
             < _fiota((_EXPERTS, _EXPERTS), 1))
    starts = jnp.dot(padded, tril8.astype(_F32),
                     preferred_element_type=_F32,
                     precision=jax.lax.Precision.HIGHEST)
    ec = ec_ref[pl.ds(c * 512, 512), :]
    er = er_ref[:, pl.ds(c * 512, 512)]
    same = (ec == er) & (_fiota((512, 512), 1) <= _fiota((512, 512), 0))
    rank = jnp.dot(same.astype(_F32), jnp.ones((512, 1), _F32),
                   preferred_element_type=_F32,
                   precision=jax.lax.Precision.HIGHEST)
    sel = (ec == e8).astype(_F32)
    prev = jnp.where(c == 0, jnp.zeros((1, _EXPERTS), _F32), cnt_ref[...])
    base = jnp.sum(sel * (starts + prev), axis=1, keepdims=True)
    pos_ref[pl.ds(c * 512, 512), :] = base + rank - 1.0
    cnt_ref[...] = prev + jnp.sum(sel, axis=0, keepdims=True)

    @pl.when(c == 0)
    def _():
        blk = _fiota((1, nb), 1) * _RB
        be = -jnp.ones((1, nb), _F32)
        for e in range(_EXPERTS):
            sel_e = (e8 == float(e)).astype(_F32)
            se = jnp.sum(starts * sel_e, axis=1, keepdims=True)
            be = be + (blk >= se).astype(_F32)
        total = jnp.sum(padded, axis=1, keepdims=True)
        be_ref[...] = be.astype(jnp.int32)
        bv_ref[...] = (blk < total).astype(jnp.int32)


def _moe_body(bi_ref, bv_ref, nx_ref, w1_ref, b1_ref, w2_ref, b2_ref,
              pa_ref, pb_ref, y_ref):
    b = pl.program_id(0)

    @pl.when(bv_ref[b] > 0)
    def _():
        n = nx_ref.shape[0]
        si = jax.lax.broadcasted_iota(jnp.int32, (n, _RB), 1) + b * _RB
        pai = pa_ref[...].astype(jnp.int32)
        pbi = pb_ref[...].astype(jnp.int32)
        hit = ((pai == si) | (pbi == si)).astype(_BF)
        xblk = jax.lax.dot_general(
            hit, nx_ref[...], (((0,), (0,)), ((), ())),
            preferred_element_type=_F32,
        ).astype(_BF)
        h1 = jnp.dot(xblk, w1_ref[0].astype(_BF),
                     preferred_element_type=_F32) + b1_ref[0]
        h1 = (h1 * 0.5 * (1.0 + jax.lax.erf(h1 * (2.0 ** -0.5)))).astype(_BF)
        y = jnp.dot(h1, w2_ref[0].astype(_BF),
                    preferred_element_type=_F32) + b2_ref[0]
        y_ref[...] = y.astype(_BF)


def _combine_body(xr_ref, pa_ref, pb_ref, ga_ref, gb_ref, yall_ref, out_ref):
    pad = yall_ref.shape[0]
    rows = xr_ref.shape[0]
    srow = jax.lax.broadcasted_iota(jnp.int32, (rows, pad), 1)
    pai = pa_ref[...].astype(jnp.int32)
    pbi = pb_ref[...].astype(jnp.int32)
    ohc = ((pai == srow).astype(_BF) * ga_ref[...].astype(_BF)
           + (pbi == srow).astype(_BF) * gb_ref[...].astype(_BF))
    out_ref[...] = xr_ref[...] + jnp.dot(
        ohc, yall_ref[...], preferred_element_type=_F32)


def kernel(x, ln1_g, ln1_b, qkv_w, qkv_b, proj_w, proj_b, ln2_g, ln2_b,
           w_gate, W1, b1, W2, b2):
    B, S, C = x.shape
    N = B * S
    xf = x.reshape(N, C)
    r1 = lambda a: a.reshape(1, -1)

    qkv = pl.pallas_call(
        _ln_qkv_body,
        grid=(N // 256,),
        in_specs=[
            pl.BlockSpec((256, C), lambda i: (i, 0)),
            pl.BlockSpec((1, C), lambda i: (0, 0)),
            pl.BlockSpec((1, C), lambda i: (0, 0)),
            pl.BlockSpec((C, 3 * C), lambda i: (0, 0)),
            pl.BlockSpec((1, 3 * C), lambda i: (0, 0)),
        ],
        out_specs=pl.BlockSpec((256, 3 * C), lambda i: (i, 0)),
        out_shape=jax.ShapeDtypeStruct((N, 3 * C), _F32),
    )(xf, r1(ln1_g), r1(ln1_b), qkv_w.astype(_BF), r1(qkv_b))

    o = pl.pallas_call(
        _attn_body,
        grid=(_HEADS // 2, N // 512),
        in_specs=[
            pl.BlockSpec((512, 2 * _DH), lambda h, qb: (qb, h)),
            pl.BlockSpec((N, 2 * _DH), lambda h, qb: (0, _HEADS // 2 + h)),
            pl.BlockSpec((N, 2 * _DH), lambda h, qb: (0, _HEADS + h)),
        ],
        out_specs=pl.BlockSpec((512, 2 * _DH), lambda h, qb: (qb, h)),
        out_shape=jax.ShapeDtypeStruct((N, C), _F32),
    )(qkv, qkv, qkv)

    xr, nxb, tidx, tg = pl.pallas_call(
        _proj_gate_body,
        grid=(N // 256,),
        in_specs=[
            pl.BlockSpec((256, C), lambda i: (i, 0)),
            pl.BlockSpec((256, C), lambda i: (i, 0)),
            pl.BlockSpec((C, C), lambda i: (0, 0)),
            pl.BlockSpec((1, C), lambda i: (0, 0)),
            pl.BlockSpec((1, C), lambda i: (0, 0)),
            pl.BlockSpec((1, C), lambda i: (0, 0)),
            pl.BlockSpec((C, _EXPERTS), lambda i: (0, 0)),
        ],
        out_specs=[
            pl.BlockSpec((256, C), lambda i: (i, 0)),
            pl.BlockSpec((256, C), lambda i: (i, 0)),
            pl.BlockSpec((256, 2), lambda i: (i, 0)),
            pl.BlockSpec((256, 2), lambda i: (i, 0)),
        ],
        out_shape=[
            jax.ShapeDtypeStruct((N, C), _F32),
            jax.ShapeDtypeStruct((N, C), _BF),
            jax.ShapeDtypeStruct((N, 2), jnp.int32),
            jax.ShapeDtypeStruct((N, 2), _F32),
        ],
    )(xf, o, proj_w.astype(_BF), r1(proj_b), r1(ln2_g), r1(ln2_b),
      w_gate.astype(_BF))

    A = 2 * N
    NB = (A + _EXPERTS * (_RB - 1) + _RB - 1) // _RB
    NC = A // 512
    ef_row = tidx.reshape(1, A).astype(_F32)
    ef_col = tidx.reshape(A, 1).astype(_F32)
    pos, be_ix, be_valid = pl.pallas_call(
        _route_a_body,
        grid=(NC,),
        in_specs=[
            pl.BlockSpec((1, A), lambda c: (0, 0)),
            pl.BlockSpec((A, 1), lambda c: (0, 0)),
        ],
        out_specs=[
            pl.BlockSpec((A, 1), lambda c: (0, 0)),
            pl.BlockSpec((1, NB), lambda c: (0, 0)),
            pl.BlockSpec((1, NB), lambda c: (0, 0)),
        ],
        out_shape=[
            jax.ShapeDtypeStruct((A, 1), _F32),
            jax.ShapeDtypeStruct((1, NB), jnp.int32),
            jax.ShapeDtypeStruct((1, NB), jnp.int32),
        ],
        scratch_shapes=[pltpu.VMEM((1, _EXPERTS), _F32)],
    )(ef_row, ef_col)
    pos2 = pos.reshape(N, 2)
    pa = pos2[:, 0:1]
    pb = pos2[:, 1:2]
    ga = tg[:, 0:1]
    gb = tg[:, 1:2]

    PAD = NB * _RB
    yall = pl.pallas_call(
        _moe_body,
        grid_spec=pltpu.PrefetchScalarGridSpec(
            num_scalar_prefetch=2,
            grid=(NB,),
            in_specs=[
                pl.BlockSpec((N, C), lambda b, bi, bv: (0, 0)),
                pl.BlockSpec((1, C, _HIDDEN), lambda b, bi, bv: (bi[b], 0, 0)),
                pl.BlockSpec((1, 1, _HIDDEN), lambda b, bi, bv: (bi[b], 0, 0)),
                pl.BlockSpec((1, _HIDDEN, C), lambda b, bi, bv: (bi[b], 0, 0)),
                pl.BlockSpec((1, 1, C), lambda b, bi, bv: (bi[b], 0, 0)),
                pl.BlockSpec((N, 1), lambda b, bi, bv: (0, 0)),
                pl.BlockSpec((N, 1), lambda b, bi, bv: (0, 0)),
            ],
            out_specs=pl.BlockSpec((_RB, C), lambda b, bi, bv: (b, 0)),
        ),
        out_shape=jax.ShapeDtypeStruct((PAD, C), _BF),
    )(be_ix.reshape(NB), be_valid.reshape(NB), nxb,
      W1, b1.reshape(_EXPERTS, 1, _HIDDEN),
      W2, b2.reshape(_EXPERTS, 1, C),
      pa, pb)

    out = pl.pallas_call(
        _combine_body,
        grid=(N // 256,),
        in_specs=[
            pl.BlockSpec((256, C), lambda i: (i, 0)),
            pl.BlockSpec((256, 1), lambda i: (i, 0)),
            pl.BlockSpec((256, 1), lambda i: (i, 0)),
            pl.BlockSpec((256, 1), lambda i: (i, 0)),
            pl.BlockSpec((256, 1), lambda i: (i, 0)),
            pl.BlockSpec((PAD, C), lambda i: (0, 0)),
        ],
        out_specs=pl.BlockSpec((256, C), lambda i: (i, 0)),
        out_shape=jax.ShapeDtypeStruct((N, C), _F32),
    )(xr, pa, pb, ga, gb, yall)

    return out.reshape(B, S, C)


# speedup vs baseline: 1.0654x; 1.0654x over previous
import jax
import jax.numpy as jnp
from jax.experimental import pallas as pl
from jax.experimental.pallas import tpu as pltpu

_HEADS = 12
_DH = 64
_EXPERTS = 8
_HIDDEN = 3072
_RB = 256
_LN_EPS = 1e-5
_SCALE = _DH ** -0.5
_NEG = -1e30
_BF = jnp.bfloat16
_F32 = jnp.float32


def _ln(xb, g, b):
    m = jnp.mean(xb, axis=-1, keepdims=True)
    v = jnp.mean((xb - m) ** 2, axis=-1, keepdims=True)
    return (xb - m) * jax.lax.rsqrt(v + _LN_EPS) * g + b


def _ln_qkv_body(x_ref, g_ref, b_ref, w_ref, wb_ref, out_ref):
    h = _ln(x_ref[...], g_ref[...], b_ref[...])
    out_ref[...] = (
        jnp.dot(h.astype(_BF), w_ref[...], preferred_element_type=_F32)
        + wb_ref[...]
    )


def _attn_body(q_ref, k_ref, v_ref, o_ref):
    q = q_ref[...].astype(_BF)
    k = k_ref[...].astype(_BF)
    v = v_ref[...].astype(_BF)
    outs = []
    for i in range(2):
        qp = q[:, i * _DH:(i + 1) * _DH]
        kp = k[:, i * _DH:(i + 1) * _DH]
        vp = v[:, i * _DH:(i + 1) * _DH]
        s = jax.lax.dot_general(
            qp, kp, (((1,), (1,)), ((), ())), preferred_element_type=_F32
        ) * _SCALE
        m = jnp.max(s, axis=-1, keepdims=True)
        p = jnp.exp(s - m)
        p = p / jnp.sum(p, axis=-1, keepdims=True)
        outs.append(jnp.dot(p.astype(_BF), vp, preferred_element_type=_F32))
    o_ref[...] = jnp.concatenate(outs, axis=-1)


def _proj_gate_body(x_ref, o_ref, pw_ref, pb_ref, g2_ref, b2_ref, wg_ref,
                    xr_ref, nx_ref, idx_ref, gate_ref):
    xr = x_ref[...] + jnp.dot(
        o_ref[...].astype(_BF), pw_ref[...], preferred_element_type=_F32
    ) + pb_ref[...]
    xr_ref[...] = xr
    nx = _ln(xr, g2_ref[...], b2_ref[...])
    nxb = nx.astype(_BF)
    nx_ref[...] = nxb
    logits = jnp.dot(nxb, wg_ref[...], preferred_element_type=_F32)
    ii = jax.lax.broadcasted_iota(jnp.int32, logits.shape, 1)
    m1 = jnp.max(logits, axis=-1, keepdims=True)
    i1 = jnp.min(jnp.where(logits == m1, ii, _EXPERTS), axis=-1, keepdims=True)
    l2 = jnp.where(ii == i1, _NEG, logits)
    m2 = jnp.max(l2, axis=-1, keepdims=True)
    i2 = jnp.min(jnp.where(l2 == m2, ii, _EXPERTS), axis=-1, keepdims=True)
    d = jnp.exp(m2 - m1)
    g1 = 1.0 / (1.0 + d)
    g2 = d / (1.0 + d)
    idx_ref[...] = jnp.concatenate([i1, i2], axis=-1)
    gate_ref[...] = jnp.concatenate([g1, g2], axis=-1)


def _fiota(shape, d):
    return jax.lax.broadcasted_iota(jnp.int32, shape, d).astype(_F32)


def _route_a_body(er_ref, ec_ref, pos_ref, be_ref, bv_ref, cnt_ref):
    c = pl.program_id(0)
    nb = be_ref.shape[1]
    e8 = _fiota((1, _EXPERTS), 1)
    onehot_all = (ec_ref[...] == e8).astype(_F32)
    counts = jnp.sum(onehot_all, axis=0, keepdims=True)
    padded = jnp.floor((counts + (_RB - 1)) * (1.0 / _RB)) * _RB
    tril8 = (_fiota((_EXPERTS, _EXPERTS), 0)
             < _fiota((_EXPERTS, _EXPERTS), 1))
    starts = jnp.dot(padded, tril8.astype(_F32),
                     preferred_element_type=_F32,
                     precision=jax.lax.Precision.HIGHEST)
    ec = ec_ref[pl.ds(c * 512, 512), :]
    er = er_ref[:, pl.ds(c * 512, 512)]
    same = (ec == er) & (_fiota((512, 512), 1) <= _fiota((512, 512), 0))
    rank = jnp.dot(same.astype(_F32), jnp.ones((512, 1), _F32),
                   preferred_element_type=_F32,
                   precision=jax.lax.Precision.HIGHEST)
    sel = (ec == e8).astype(_F32)
    prev = jnp.where(c == 0, jnp.zeros((1, _EXPERTS), _F32), cnt_ref[...])
    base = jnp.sum(sel * (starts + prev), axis=1, keepdims=True)
    pos_ref[pl.ds(c * 512, 512), :] = base + rank - 1.0
    cnt_ref[...] = prev + jnp.sum(sel, axis=0, keepdims=True)

    @pl.when(c == 0)
    def _():
        blk = _fiota((1, nb), 1) * _RB
        be = -jnp.ones((1, nb), _F32)
        for e in range(_EXPERTS):
            sel_e = (e8 == float(e)).astype(_F32)
            se = jnp.sum(starts * sel_e, axis=1, keepdims=True)
            be = be + (blk >= se).astype(_F32)
        total = jnp.sum(padded, axis=1, keepdims=True)
        be_ref[...] = be.astype(jnp.int32)
        bv_ref[...] = (blk < total).astype(jnp.int32)


def _moe_body(bi_ref, bv_ref, nx_ref, w1_ref, b1_ref, w2_ref, b2_ref,
              pa_ref, pb_ref, ga_ref, gb_ref, out_ref):
    b = pl.program_id(0)

    @pl.when(b == 0)
    def _():
        out_ref[...] = jnp.zeros_like(out_ref)

    @pl.when(bv_ref[b] > 0)
    def _():
        n = nx_ref.shape[0]
        si = jax.lax.broadcasted_iota(jnp.int32, (n, _RB), 1) + b * _RB
        pai = pa_ref[...].astype(jnp.int32)
        pbi = pb_ref[...].astype(jnp.int32)
        hita = (pai == si).astype(_BF)
        hitb = (pbi == si).astype(_BF)
        xblk = jax.lax.dot_general(
            hita + hitb, nx_ref[...], (((0,), (0,)), ((), ())),
            preferred_element_type=_F32,
        ).astype(_BF)
        h1 = jnp.dot(xblk, w1_ref[0].astype(_BF),
                     preferred_element_type=_F32) + b1_ref[0]
        h1 = (h1 * 0.5 * (1.0 + jax.lax.erf(h1 * (2.0 ** -0.5)))).astype(_BF)
        y = jnp.dot(h1, w2_ref[0].astype(_BF),
                    preferred_element_type=_F32) + b2_ref[0]
        ohg = (hita * ga_ref[...].astype(_BF)
               + hitb * gb_ref[...].astype(_BF))
        out_ref[...] += jnp.dot(ohg, y.astype(_BF), preferred_element_type=_F32)


def kernel(x, ln1_g, ln1_b, qkv_w, qkv_b, proj_w, proj_b, ln2_g, ln2_b,
           w_gate, W1, b1, W2, b2):
    B, S, C = x.shape
    N = B * S
    xf = x.reshape(N, C)
    r1 = lambda a: a.reshape(1, -1)

    qkv = pl.pallas_call(
        _ln_qkv_body,
        grid=(N // 256,),
        in_specs=[
            pl.BlockSpec((256, C), lambda i: (i, 0)),
            pl.BlockSpec((1, C), lambda i: (0, 0)),
            pl.BlockSpec((1, C), lambda i: (0, 0)),
            pl.BlockSpec((C, 3 * C), lambda i: (0, 0)),
            pl.BlockSpec((1, 3 * C), lambda i: (0, 0)),
        ],
        out_specs=pl.BlockSpec((256, 3 * C), lambda i: (i, 0)),
        out_shape=jax.ShapeDtypeStruct((N, 3 * C), _F32),
    )(xf, r1(ln1_g), r1(ln1_b), qkv_w.astype(_BF), r1(qkv_b))

    o = pl.pallas_call(
        _attn_body,
        grid=(_HEADS // 2, N // 1024),
        in_specs=[
            pl.BlockSpec((1024, 2 * _DH), lambda h, qb: (qb, h)),
            pl.BlockSpec((N, 2 * _DH), lambda h, qb: (0, _HEADS // 2 + h)),
            pl.BlockSpec((N, 2 * _DH), lambda h, qb: (0, _HEADS + h)),
        ],
        out_specs=pl.BlockSpec((1024, 2 * _DH), lambda h, qb: (qb, h)),
        out_shape=jax.ShapeDtypeStruct((N, C), _F32),
    )(qkv, qkv, qkv)

    xr, nxb, tidx, tg = pl.pallas_call(
        _proj_gate_body,
        grid=(N // 256,),
        in_specs=[
            pl.BlockSpec((256, C), lambda i: (i, 0)),
            pl.BlockSpec((256, C), lambda i: (i, 0)),
            pl.BlockSpec((C, C), lambda i: (0, 0)),
            pl.BlockSpec((1, C), lambda i: (0, 0)),
            pl.BlockSpec((1, C), lambda i: (0, 0)),
            pl.BlockSpec((1, C), lambda i: (0, 0)),
            pl.BlockSpec((C, _EXPERTS), lambda i: (0, 0)),
        ],
        out_specs=[
            pl.BlockSpec((256, C), lambda i: (i, 0)),
            pl.BlockSpec((256, C), lambda i: (i, 0)),
            pl.BlockSpec((256, 2), lambda i: (i, 0)),
            pl.BlockSpec((256, 2), lambda i: (i, 0)),
        ],
        out_shape=[
            jax.ShapeDtypeStruct((N, C), _F32),
            jax.ShapeDtypeStruct((N, C), _BF),
            jax.ShapeDtypeStruct((N, 2), jnp.int32),
            jax.ShapeDtypeStruct((N, 2), _F32),
        ],
    )(xf, o, proj_w.astype(_BF), r1(proj_b), r1(ln2_g), r1(ln2_b),
      w_gate.astype(_BF))

    A = 2 * N
    NB = (A + _EXPERTS * (_RB - 1) + _RB - 1) // _RB
    NC = A // 512
    ef_row = tidx.reshape(1, A).astype(_F32)
    ef_col = tidx.reshape(A, 1).astype(_F32)
    pos, be_ix, be_valid = pl.pallas_call(
        _route_a_body,
        grid=(NC,),
        in_specs=[
            pl.BlockSpec((1, A), lambda c: (0, 0)),
            pl.BlockSpec((A, 1), lambda c: (0, 0)),
        ],
        out_specs=[
            pl.BlockSpec((A, 1), lambda c: (0, 0)),
            pl.BlockSpec((1, NB), lambda c: (0, 0)),
            pl.BlockSpec((1, NB), lambda c: (0, 0)),
        ],
        out_shape=[
            jax.ShapeDtypeStruct((A, 1), _F32),
            jax.ShapeDtypeStruct((1, NB), jnp.int32),
            jax.ShapeDtypeStruct((1, NB), jnp.int32),
        ],
        scratch_shapes=[pltpu.VMEM((1, _EXPERTS), _F32)],
    )(ef_row, ef_col)
    pos2 = pos.reshape(N, 2)
    pa = pos2[:, 0:1]
    pb = pos2[:, 1:2]
    ga = tg[:, 0:1]
    gb = tg[:, 1:2]

    out = pl.pallas_call(
        _moe_body,
        grid_spec=pltpu.PrefetchScalarGridSpec(
            num_scalar_prefetch=2,
            grid=(NB,),
            in_specs=[
                pl.BlockSpec((N, C), lambda b, bi, bv: (0, 0)),
                pl.BlockSpec((1, C, _HIDDEN), lambda b, bi, bv: (bi[b], 0, 0)),
                pl.BlockSpec((1, 1, _HIDDEN), lambda b, bi, bv: (bi[b], 0, 0)),
                pl.BlockSpec((1, _HIDDEN, C), lambda b, bi, bv: (bi[b], 0, 0)),
                pl.BlockSpec((1, 1, C), lambda b, bi, bv: (bi[b], 0, 0)),
                pl.BlockSpec((N, 1), lambda b, bi, bv: (0, 0)),
                pl.BlockSpec((N, 1), lambda b, bi, bv: (0, 0)),
                pl.BlockSpec((N, 1), lambda b, bi, bv: (0, 0)),
                pl.BlockSpec((N, 1), lambda b, bi, bv: (0, 0)),
            ],
            out_specs=pl.BlockSpec((N, C), lambda b, bi, bv: (0, 0)),
        ),
        out_shape=jax.ShapeDtypeStruct((N, C), _F32),
    )(be_ix.reshape(NB), be_valid.reshape(NB), nxb,
      W1, b1.reshape(_EXPERTS, 1, _HIDDEN),
      W2, b2.reshape(_EXPERTS, 1, C),
      pa, pb, ga, gb)

    return (xr + out).reshape(B, S, C)
